# baseline (speedup 1.0000x reference)
import jax
import jax.numpy as jnp
from jax.experimental import pallas as pl
from jax.experimental.pallas import tpu as pltpu

C = 128
K = 4
R0 = 8
MM = jnp.bfloat16


def _fused_kernel(x_ref, w1_ref, b1_ref, w2_ref, b2_ref, w3_ref, b3_ref,
                  w4_ref, b4_ref, wf1_ref, bf1_ref, wf2_ref, bf2_ref,
                  o_ref, col_ref, flat_ref):
    L, Bb, _ = x_ref.shape
    f32 = jnp.float32

    col_ref[R0:R0 + 1, :, 0:C] = jnp.zeros((1, Bb, C), MM)
    col_ref[R0 + L - 1:R0 + L, :, 2 * C:3 * C] = jnp.zeros((1, Bb, C), MM)
    col_ref[R0 + L - 2:R0 + L, :, 3 * C:4 * C] = jnp.zeros((2, Bb, C), MM)

    def conv_block(h, w_ref, b_ref):
        hb = h.astype(MM)
        for j in range(K):
            col_ref[R0 + 1 - j:R0 + 1 - j + L, :, j * C:(j + 1) * C] = hb
        y = jax.lax.dot_general(
            col_ref[R0:R0 + L, :, :], w_ref[...],
            dimension_numbers=(((2,), (0,)), ((), ())),
            preferred_element_type=f32)
        return jnp.maximum(y + b_ref[...], 0.0)

    xb = jax.lax.dot_general(
        x_ref[...].astype(MM), jnp.ones((1, C), MM),
        dimension_numbers=(((2,), (0,)), ((), ())),
        preferred_element_type=f32)
    h = conv_block(xb, w1_ref, b1_ref)
    h = conv_block(h, w2_ref, b2_ref)
    h = conv_block(h, w3_ref, b3_ref)
    h = conv_block(h, w4_ref, b4_ref)

    hb = h.astype(MM)
    for t in range(L):
        flat_ref[:, t * C:(t + 1) * C] = hb[t]
    z = jnp.dot(flat_ref[...], wf1_ref[...],
                preferred_element_type=f32) + bf1_ref[...]
    z = jnp.maximum(z, 0.0)
    o_ref[...] = jnp.sum(z * wf2_ref[...], axis=-1, keepdims=True) + bf2_ref[...]


def kernel(x_ncl, w1c, b1c, w2c, b2c, w3c, b3c, w4c, b4c, wf1, bf1, wf2, bf2):
    B, cin0, L = x_ncl.shape
    x_lb = jnp.transpose(x_ncl, (2, 0, 1)).astype(jnp.float32)

    bb = 32
    grid = (B // bb,)

    w1e = jnp.pad(w1c, ((0, 0), (0, C - 1), (0, 0))).reshape(K * C, C)
    weights = [w1e.astype(MM), b1c, w2c.astype(MM), b2c, w3c.astype(MM),
               b3c, w4c.astype(MM), b4c, wf1.astype(MM), bf1, wf2, bf2]

    def rep_spec(shape):
        n = len(shape)
        return pl.BlockSpec(shape, lambda i, n=n: (0,) * n)

    in_specs = ([pl.BlockSpec((L, bb, 1), lambda i: (0, i, 0))]
                + [rep_spec(w.shape) for w in weights])
    out_specs = pl.BlockSpec((bb, 1), lambda i: (i, 0))

    return pl.pallas_call(
        _fused_kernel,
        out_shape=jax.ShapeDtypeStruct((B, 1), jnp.float32),
        grid=grid,
        in_specs=in_specs,
        out_specs=out_specs,
        scratch_shapes=[
            pltpu.VMEM((L + 16, bb, K * C), MM),
            pltpu.VMEM((bb, L * C), MM),
        ],
        compiler_params=pltpu.CompilerParams(dimension_semantics=("parallel",)),
    )(x_lb, *weights)


# speedup vs baseline: 1.3918x; 1.3918x over previous
import jax
import jax.numpy as jnp
from jax.experimental import pallas as pl
from jax.experimental.pallas import tpu as pltpu

C = 128
K = 4
R0 = 8
MM = jnp.bfloat16


def _fused_kernel(x_ref, w1_ref, b1_ref, w2_ref, b2_ref, w3_ref, b3_ref,
                  w4_ref, b4_ref, wf1_ref, bf1_ref, wf2_ref, bf2_ref,
                  o_ref, col_ref, flat_ref):
    _, L, Bb, _ = x_ref.shape
    f32 = jnp.float32

    col_ref[R0:R0 + 1, :, 0:C] = jnp.zeros((1, Bb, C), MM)
    col_ref[R0 + L - 1:R0 + L, :, 2 * C:3 * C] = jnp.zeros((1, Bb, C), MM)
    col_ref[R0 + L - 2:R0 + L, :, 3 * C:4 * C] = jnp.zeros((2, Bb, C), MM)

    def conv_block(h, w_ref, b_ref):
        hb = h.astype(MM)
        for j in range(K):
            col_ref[R0 + 1 - j:R0 + 1 - j + L, :, j * C:(j + 1) * C] = hb
        y = jax.lax.dot_general(
            col_ref[R0:R0 + L, :, :], w_ref[...],
            dimension_numbers=(((2,), (0,)), ((), ())),
            preferred_element_type=f32)
        return jnp.maximum(y + b_ref[...], 0.0)

    h = conv_block(x_ref[0], w1_ref, b1_ref)
    h = conv_block(h, w2_ref, b2_ref)
    h = conv_block(h, w3_ref, b3_ref)
    h = conv_block(h, w4_ref, b4_ref)

    hb = h.astype(MM)
    for t in range(L):
        flat_ref[:, t * C:(t + 1) * C] = hb[t]
    z = jnp.dot(flat_ref[...], wf1_ref[...],
                preferred_element_type=f32) + bf1_ref[...]
    z = jnp.maximum(z, 0.0)
    o_ref[...] = jnp.sum(z * wf2_ref[...], axis=-1, keepdims=True) + bf2_ref[...]


def kernel(x_ncl, w1c, b1c, w2c, b2c, w3c, b3c, w4c, b4c, wf1, bf1, wf2, bf2):
    B, cin0, L = x_ncl.shape
    bb = 32
    x_lb = jnp.broadcast_to(
        jnp.transpose(x_ncl.reshape(B // bb, bb, L).astype(MM),
                      (0, 2, 1))[..., None],
        (B // bb, L, bb, C))

    grid = (B // bb,)

    w1e = jnp.pad(w1c, ((0, 0), (0, C - 1), (0, 0))).reshape(K * C, C)
    weights = [w1e.astype(MM), b1c, w2c.astype(MM), b2c, w3c.astype(MM),
               b3c, w4c.astype(MM), b4c, wf1.astype(MM), bf1, wf2, bf2]

    def rep_spec(shape):
        n = len(shape)
        return pl.BlockSpec(shape, lambda i, n=n: (0,) * n)

    in_specs = ([pl.BlockSpec((1, L, bb, C), lambda i: (i, 0, 0, 0))]
                + [rep_spec(w.shape) for w in weights])
    out_specs = pl.BlockSpec((bb, 1), lambda i: (i, 0))

    return pl.pallas_call(
        _fused_kernel,
        out_shape=jax.ShapeDtypeStruct((B, 1), jnp.float32),
        grid=grid,
        in_specs=in_specs,
        out_specs=out_specs,
        scratch_shapes=[
            pltpu.VMEM((L + 16, bb, K * C), MM),
            pltpu.VMEM((bb, L * C), MM),
        ],
        compiler_params=pltpu.CompilerParams(dimension_semantics=("parallel",)),
    )(x_lb, *weights)


# speedup vs baseline: 1.4694x; 1.0558x over previous
import jax
import jax.numpy as jnp
from jax.experimental import pallas as pl
from jax.experimental.pallas import tpu as pltpu

C = 128
K = 4
R0 = 8
MM = jnp.bfloat16


def _fused_kernel(x_ref, w1_ref, b1_ref, w2_ref, b2_ref, w3_ref, b3_ref,
                  w4_ref, b4_ref, wf1_ref, bf1_ref, wf2_ref, bf2_ref,
                  o_ref, col_ref, flat_ref):
    _, L, Bb, _ = x_ref.shape
    f32 = jnp.float32

    col_ref[R0:R0 + 1, :, 0:C] = jnp.zeros((1, Bb, C), MM)
    col_ref[R0 + L - 1:R0 + L, :, 2 * C:3 * C] = jnp.zeros((1, Bb, C), MM)
    col_ref[R0 + L - 2:R0 + L, :, 3 * C:4 * C] = jnp.zeros((2, Bb, C), MM)

    def conv_block(h, w_ref, b_ref):
        hb = h.astype(MM)
        for j in range(K):
            col_ref[R0 + 1 - j:R0 + 1 - j + L, :, j * C:(j + 1) * C] = hb
        y = jax.lax.dot_general(
            col_ref[R0:R0 + L, :, :], w_ref[...],
            dimension_numbers=(((2,), (0,)), ((), ())),
            preferred_element_type=f32)
        return jnp.maximum(y + b_ref[...], 0.0)

    h = conv_block(x_ref[0], w1_ref, b1_ref)
    h = conv_block(h, w2_ref, b2_ref)
    h = conv_block(h, w3_ref, b3_ref)
    h = conv_block(h, w4_ref, b4_ref)

    hb = h.astype(MM)
    for t in range(L):
        flat_ref[:, t * C:(t + 1) * C] = hb[t]
    z = jnp.dot(flat_ref[...], wf1_ref[...],
                preferred_element_type=f32) + bf1_ref[...]
    z = jnp.maximum(z, 0.0)
    o_ref[...] = jnp.sum(z * wf2_ref[...], axis=-1, keepdims=True) + bf2_ref[...]


def kernel(x_ncl, w1c, b1c, w2c, b2c, w3c, b3c, w4c, b4c, wf1, bf1, wf2, bf2):
    B, cin0, L = x_ncl.shape
    bb = 64
    x_lb = jnp.broadcast_to(
        jnp.transpose(x_ncl.reshape(B // bb, bb, L).astype(MM),
                      (0, 2, 1))[..., None],
        (B // bb, L, bb, C))

    grid = (B // bb,)

    w1e = jnp.pad(w1c, ((0, 0), (0, C - 1), (0, 0))).reshape(K * C, C)
    weights = [w1e.astype(MM), b1c, w2c.astype(MM), b2c, w3c.astype(MM),
               b3c, w4c.astype(MM), b4c, wf1.astype(MM), bf1, wf2, bf2]

    def rep_spec(shape):
        n = len(shape)
        return pl.BlockSpec(shape, lambda i, n=n: (0,) * n)

    in_specs = ([pl.BlockSpec((1, L, bb, C), lambda i: (i, 0, 0, 0))]
                + [rep_spec(w.shape) for w in weights])
    out_specs = pl.BlockSpec((bb, 1), lambda i: (i, 0))

    return pl.pallas_call(
        _fused_kernel,
        out_shape=jax.ShapeDtypeStruct((B, 1), jnp.float32),
        grid=grid,
        in_specs=in_specs,
        out_specs=out_specs,
        scratch_shapes=[
            pltpu.VMEM((L + 16, bb, K * C), MM),
            pltpu.VMEM((bb, L * C), MM),
        ],
        compiler_params=pltpu.CompilerParams(dimension_semantics=("parallel",)),
    )(x_lb, *weights)


# speedup vs baseline: 1.4814x; 1.0082x over previous
import jax
import jax.numpy as jnp
from jax.experimental import pallas as pl
from jax.experimental.pallas import tpu as pltpu

C = 128
K = 4
R0 = 8
XP = 8


def _fused_kernel(x_ref, w1_ref, b1_ref, w2_ref, b2_ref, w3_ref, b3_ref,
                  w4_ref, b4_ref, wf1_ref, bf1_ref, wf2_ref, bf2_ref,
                  o_ref, col_ref, flat_ref):
    _, Lp, Bb, _ = x_ref.shape
    L = Lp - XP
    f32 = jnp.float32

    col_ref[R0:R0 + 1, :, 0:C] = jnp.zeros((1, Bb, C), f32)
    col_ref[R0 + L - 1:R0 + L, :, 2 * C:3 * C] = jnp.zeros((1, Bb, C), f32)
    col_ref[R0 + L - 2:R0 + L, :, 3 * C:4 * C] = jnp.zeros((2, Bb, C), f32)

    y = None
    for j in range(K):
        p = jax.lax.dot_general(
            x_ref[0, j:j + L], w1_ref[j * C:(j + 1) * C, :],
            dimension_numbers=(((2,), (0,)), ((), ())),
            preferred_element_type=f32)
        y = p if y is None else y + p
    h = jnp.maximum(y + b1_ref[...], 0.0)

    def conv_block(h, w_ref, b_ref):
        for j in range(K):
            col_ref[R0 + 1 - j:R0 + 1 - j + L, :, j * C:(j + 1) * C] = h
        y = jax.lax.dot_general(
            col_ref[R0:R0 + L, :, :], w_ref[...],
            dimension_numbers=(((2,), (0,)), ((), ())),
            preferred_element_type=f32)
        return jnp.maximum(y + b_ref[...], 0.0)

    h = conv_block(h, w2_ref, b2_ref)
    h = conv_block(h, w3_ref, b3_ref)
    h = conv_block(h, w4_ref, b4_ref)

    for t in range(L):
        flat_ref[:, t * C:(t + 1) * C] = h[t]
    z = jnp.dot(flat_ref[...], wf1_ref[...],
                preferred_element_type=f32) + bf1_ref[...]
    z = jnp.maximum(z, 0.0)
    o_ref[...] = jnp.sum(z * wf2_ref[...], axis=-1, keepdims=True) + bf2_ref[...]


def kernel(x_ncl, w1c, b1c, w2c, b2c, w3c, b3c, w4c, b4c, wf1, bf1, wf2, bf2):
    B, cin0, L = x_ncl.shape
    bf16 = jnp.bfloat16
    bb = 32

    xt = jnp.transpose(x_ncl.reshape(B // bb, bb, L).astype(bf16), (0, 2, 1))
    xt = jnp.pad(xt, ((0, 0), (1, XP - 1), (0, 0)))
    x_lb = jnp.broadcast_to(xt[..., None], (B // bb, L + XP, bb, C))

    grid = (B // bb,)

    w1e = jnp.pad(w1c, ((0, 0), (0, C - 1), (0, 0))).reshape(K * C, C)
    weights = [w1e.astype(bf16), b1c, w2c, b2c, w3c, b3c, w4c, b4c,
               wf1, bf1, wf2, bf2]

    def rep_spec(shape):
        n = len(shape)
        return pl.BlockSpec(shape, lambda i, n=n: (0,) * n)

    in_specs = ([pl.BlockSpec((1, L + XP, bb, C), lambda i: (i, 0, 0, 0))]
                + [rep_spec(w.shape) for w in weights])
    out_specs = pl.BlockSpec((bb, 1), lambda i: (i, 0))

    return pl.pallas_call(
        _fused_kernel,
        out_shape=jax.ShapeDtypeStruct((B, 1), jnp.float32),
        grid=grid,
        in_specs=in_specs,
        out_specs=out_specs,
        scratch_shapes=[
            pltpu.VMEM((L + 16, bb, K * C), jnp.float32),
            pltpu.VMEM((bb, L * C), jnp.float32),
        ],
        compiler_params=pltpu.CompilerParams(dimension_semantics=("parallel",)),
    )(x_lb, *weights)


# speedup vs baseline: 1.8848x; 1.2723x over previous
import jax
import jax.numpy as jnp
from jax.experimental import pallas as pl
from jax.experimental.pallas import tpu as pltpu

C = 128
K = 4
R0 = 8
XP = 8


def _fused_kernel(x_ref, w1_ref, b1_ref, w2_ref, b2_ref, w3_ref, b3_ref,
                  w4_ref, b4_ref, wf1_ref, bf1_ref, wf2_ref, bf2_ref,
                  o_ref, col_ref, flat_ref):
    _, Lp, Bb, _ = x_ref.shape
    L = Lp - XP
    f32 = jnp.float32

    y = None
    for j in range(K):
        p = jax.lax.dot_general(
            x_ref[0, j:j + L], w1_ref[j * C:(j + 1) * C, :],
            dimension_numbers=(((2,), (0,)), ((), ())),
            preferred_element_type=f32)
        y = p if y is None else y + p
    h = jnp.maximum(y + b1_ref[...], 0.0)

    def conv_block(h, w_ref, b_ref, cw):
        for j in range(K):
            col_ref[R0 + 1 - j:R0 + 1 - j + L, :, j * cw:(j + 1) * cw] = h
        col_ref[R0:R0 + 1, :, 0:cw] = jnp.zeros((1, Bb, cw), f32)
        col_ref[R0 + L - 1:R0 + L, :, 2 * cw:3 * cw] = jnp.zeros((1, Bb, cw), f32)
        col_ref[R0 + L - 2:R0 + L, :, 3 * cw:4 * cw] = jnp.zeros((2, Bb, cw), f32)
        y = jax.lax.dot_general(
            col_ref[R0:R0 + L, :, 0:K * cw], w_ref[...],
            dimension_numbers=(((2,), (0,)), ((), ())),
            preferred_element_type=f32)
        return jnp.maximum(y + b_ref[...], 0.0)

    h = conv_block(h, w2_ref, b2_ref, C)
    h = conv_block(h, w3_ref, b3_ref, C)
    h = conv_block(h, w4_ref, b4_ref, 64)

    for t in range(L):
        flat_ref[:, t * 32:(t + 1) * 32] = h[t]
    z = jnp.dot(flat_ref[...], wf1_ref[...],
                preferred_element_type=f32) + bf1_ref[...]
    z = jnp.maximum(z, 0.0)
    o_ref[...] = jnp.sum(z * wf2_ref[...], axis=-1, keepdims=True) + bf2_ref[...]


def kernel(x_ncl, w1c, b1c, w2c, b2c, w3c, b3c, w4c, b4c, wf1, bf1, wf2, bf2):
    B, cin0, L = x_ncl.shape
    bf16 = jnp.bfloat16
    bb = 64

    xt = jnp.transpose(x_ncl.reshape(B // bb, bb, L).astype(bf16), (0, 2, 1))
    xt = jnp.pad(xt, ((0, 0), (1, XP - 1), (0, 0)))
    x_lb = jnp.broadcast_to(xt[..., None], (B // bb, L + XP, bb, C))

    grid = (B // bb,)

    w1e = jnp.pad(w1c, ((0, 0), (0, C - 1), (0, 0))).reshape(K * C, C)
    w3d = w3c[:, :64]
    b3d = b3c[:, :64]
    w4d = w4c.reshape(K, C, C)[:, :64, :32].reshape(K * 64, 32)
    b4d = b4c[:, :32]
    wf1d = wf1.reshape(L, C, 64)[:, :32, :].reshape(L * 32, 64)
    weights = [w1e.astype(bf16), b1c, w2c, b2c, w3d, b3d, w4d, b4d,
               wf1d, bf1, wf2, bf2]

    def rep_spec(shape):
        n = len(shape)
        return pl.BlockSpec(shape, lambda i, n=n: (0,) * n)

    in_specs = ([pl.BlockSpec((1, L + XP, bb, C), lambda i: (i, 0, 0, 0))]
                + [rep_spec(w.shape) for w in weights])
    out_specs = pl.BlockSpec((bb, 1), lambda i: (i, 0))

    return pl.pallas_call(
        _fused_kernel,
        out_shape=jax.ShapeDtypeStruct((B, 1), jnp.float32),
        grid=grid,
        in_specs=in_specs,
        out_specs=out_specs,
        scratch_shapes=[
            pltpu.VMEM((L + 16, bb, K * C), jnp.float32),
            pltpu.VMEM((bb, L * 32), jnp.float32),
        ],
        compiler_params=pltpu.CompilerParams(dimension_semantics=("parallel",)),
    )(x_lb, *weights)


# speedup vs baseline: 1.9950x; 1.0584x over previous
import jax
import jax.numpy as jnp
from jax.experimental import pallas as pl
from jax.experimental.pallas import tpu as pltpu

C = 128
K = 4
R0 = 8
XP = 8


def _fused_kernel(x_ref, w1_ref, b1_ref, w2_ref, b2_ref, w3_ref, b3_ref,
                  w4_ref, b4_ref, wf1_ref, bf1_ref, wf2_ref, bf2_ref,
                  o_ref, col_ref, colx_ref, flat_ref):
    _, Lp, Bb, _ = x_ref.shape
    L = Lp - XP
    f32 = jnp.float32

    def paired_conv(cref, h, w_ref, b_ref, cw):
        cref[R0 + 1:R0 + 1 + L, :, 0:cw] = h
        cref[R0:R0 + L, :, cw:2 * cw] = h
        z1 = jnp.zeros((1, Bb, cw), cref.dtype)
        cref[R0:R0 + 1, :, 0:cw] = z1
        cref[R0 + L + 1:R0 + L + 2, :, 0:cw] = z1
        cref[R0 + L:R0 + L + 2, :, cw:2 * cw] = jnp.concatenate([z1, z1], 0)
        ya = jax.lax.dot_general(
            cref[R0:R0 + L, :, 0:2 * cw], w_ref[0:2 * cw, :],
            dimension_numbers=(((2,), (0,)), ((), ())),
            preferred_element_type=f32)
        yb = jax.lax.dot_general(
            cref[R0 + 2:R0 + 2 + L, :, 0:2 * cw], w_ref[2 * cw:4 * cw, :],
            dimension_numbers=(((2,), (0,)), ((), ())),
            preferred_element_type=f32)
        return jnp.maximum(ya + yb + b_ref[...], 0.0)

    h = paired_conv(colx_ref, x_ref[0, 1:L + 1], w1_ref, b1_ref, C)
    h = paired_conv(col_ref, h, w2_ref, b2_ref, C)
    h = paired_conv(col_ref, h, w3_ref, b3_ref, C)
    h = paired_conv(col_ref, h, w4_ref, b4_ref, 64)

    for t in range(L):
        flat_ref[:, t * 32:(t + 1) * 32] = h[t]
    z = jnp.dot(flat_ref[...], wf1_ref[...],
                preferred_element_type=f32) + bf1_ref[...]
    z = jnp.maximum(z, 0.0)
    o_ref[...] = jnp.sum(z * wf2_ref[...], axis=-1, keepdims=True) + bf2_ref[...]


def kernel(x_ncl, w1c, b1c, w2c, b2c, w3c, b3c, w4c, b4c, wf1, bf1, wf2, bf2):
    B, cin0, L = x_ncl.shape
    bf16 = jnp.bfloat16
    bb = 64

    xt = jnp.transpose(x_ncl.reshape(B // bb, bb, L).astype(bf16), (0, 2, 1))
    xt = jnp.pad(xt, ((0, 0), (1, XP - 1), (0, 0)))
    x_lb = jnp.broadcast_to(xt[..., None], (B // bb, L + XP, bb, C))

    grid = (B // bb,)

    w1e = jnp.pad(w1c, ((0, 0), (0, C - 1), (0, 0))).reshape(K * C, C)
    w3d = w3c[:, :64]
    b3d = b3c[:, :64]
    w4d = w4c.reshape(K, C, C)[:, :64, :32].reshape(K * 64, 32)
    b4d = b4c[:, :32]
    wf1d = wf1.reshape(L, C, 64)[:, :32, :].reshape(L * 32, 64)
    weights = [w1e.astype(bf16), b1c, w2c, b2c, w3d, b3d, w4d, b4d,
               wf1d, bf1, wf2, bf2]

    def rep_spec(shape):
        n = len(shape)
        return pl.BlockSpec(shape, lambda i, n=n: (0,) * n)

    in_specs = ([pl.BlockSpec((1, L + XP, bb, C), lambda i: (i, 0, 0, 0))]
                + [rep_spec(w.shape) for w in weights])
    out_specs = pl.BlockSpec((bb, 1), lambda i: (i, 0))

    return pl.pallas_call(
        _fused_kernel,
        out_shape=jax.ShapeDtypeStruct((B, 1), jnp.float32),
        grid=grid,
        in_specs=in_specs,
        out_specs=out_specs,
        scratch_shapes=[
            pltpu.VMEM((L + 16, bb, 2 * C), jnp.float32),
            pltpu.VMEM((L + 16, bb, 2 * C), bf16),
            pltpu.VMEM((bb, L * 32), jnp.float32),
        ],
        compiler_params=pltpu.CompilerParams(dimension_semantics=("parallel",)),
    )(x_lb, *weights)
